# speedup vs baseline: 0.9985x
import jax
import jax.numpy as jnp
from jax.experimental import pallas as pl

N = 100000
E = 1600000
D = 128
H = 16
L = 16
G = 64
OUT = 1
NEG = 0.2


def _gat_layer(h, W, asrc, adst, b, src, dst, n):
    hp = h @ W
    logits = jax.nn.leaky_relu(
        jnp.take(hp @ asrc, src) + jnp.take(hp @ adst, dst), negative_slope=NEG)
    m = jax.ops.segment_max(logits, dst, num_segments=n)
    ex = jnp.exp(logits - jnp.take(m, dst))
    denom = jax.ops.segment_sum(ex, dst, num_segments=n)
    alpha = ex / (jnp.take(denom, dst) + 1e-16)
    out = jax.ops.segment_sum(jnp.take(hp, src, axis=0) * alpha[:, None], dst,
                              num_segments=n)
    return out + b


def _bn_relu(h, g, bta):
    mu = jnp.mean(h, axis=0)
    var = jnp.var(h, axis=0)
    return jax.nn.relu((h - mu) / jnp.sqrt(var + 1e-5) * g + bta)


def _mlp_kernel(pooled_ref, m1_ref, b1_ref, m2_ref, b2_ref, m3_ref, b3_ref,
                out_ref):
    z = jnp.maximum(pooled_ref[...] @ m1_ref[...] + b1_ref[...], 0.0)
    z = jnp.maximum(z @ m2_ref[...] + b2_ref[...], 0.0)
    out_ref[...] = z @ m3_ref[...] + b3_ref[...]


def kernel(x, edge_index, batch, W0, Wh, a_src, a_dst, bias, gamma, beta,
           M1, b1, M2, b2, M3, b3):
    n = x.shape[0]
    loops = jnp.arange(n, dtype=edge_index.dtype)
    src = jnp.concatenate([edge_index[0], loops])
    dst = jnp.concatenate([edge_index[1], loops])
    h = x
    for l in range(L):
        W = W0 if l == 0 else Wh[l - 1]
        h = _gat_layer(h, W, a_src[l], a_dst[l], bias[l], src, dst, n)
        h = _bn_relu(h, gamma[l], beta[l])
    sums = jax.ops.segment_sum(h, batch, num_segments=G)
    cnt = jax.ops.segment_sum(jnp.ones((n,), h.dtype), batch, num_segments=G)
    pooled = sums / jnp.maximum(cnt, 1.0)[:, None]
    out = pl.pallas_call(
        _mlp_kernel,
        out_shape=jax.ShapeDtypeStruct((G, OUT), jnp.float32),
    )(pooled, M1, b1, M2, b2, M3, b3)
    return out


# speedup vs baseline: 81.6521x; 81.6521x over previous
import functools

import jax
import jax.numpy as jnp
from jax import lax
from jax.experimental import pallas as pl
from jax.experimental.pallas import tpu as pltpu
from jax.experimental.pallas import tpu_sc as plsc

N = 100000
E = 1600000
D = 128
H = 16
L = 16
G = 64
OUT = 1
NEG = 0.2

NC = 2
NS = 16
NW = NC * NS

NP = 100096
RPT = NP // NS
EPW = E // NW
C = 400
NCHUNK = EPW // C

_mesh = plsc.VectorSubcoreMesh(core_axis_name="c", subcore_axis_name="s")


@functools.partial(
    pl.kernel,
    out_type=(jax.ShapeDtypeStruct((NC, NP, H), jnp.float32),
              jax.ShapeDtypeStruct((NC, NP), jnp.float32)),
    mesh=_mesh,
    compiler_params=pltpu.CompilerParams(use_tc_tiling_on_sc=False),
    scratch_types=dict(
        acc_sh=pltpu.VMEM_SHARED((NP, H), jnp.float32),
        den_sh=pltpu.VMEM_SHARED((NP,), jnp.float32),
        sidx=pltpu.VMEM((C,), jnp.int32),
        didx=pltpu.VMEM((C,), jnp.int32),
        rows=pltpu.VMEM((C, H), jnp.float32),
        sv=pltpu.VMEM((C,), jnp.float32),
        dv=pltpu.VMEM((C,), jnp.float32),
        exbuf=pltpu.VMEM((C,), jnp.float32),
        sem=pltpu.SemaphoreType.DMA,
        sem2=pltpu.SemaphoreType.DMA,
        sem3=pltpu.SemaphoreType.DMA,
    ),
)
def _edge_pass(src_hbm, dst_hbm, hp_hbm, ssrc_hbm, sdst_hbm, out_acc, out_den,
               acc_sh, den_sh, sidx, didx, rows, sv, dv, exbuf,
               sem, sem2, sem3):
    c = lax.axis_index("c")
    s = lax.axis_index("s")
    w = c * NS + s

    def zrow(i, _):
        rows[i, :] = jnp.zeros((H,), jnp.float32)
        return 0
    lax.fori_loop(0, C, zrow, 0)

    def zrowd(i, _):
        exbuf[pl.ds(i * 16, 16)] = jnp.zeros((16,), jnp.float32)
        return 0
    lax.fori_loop(0, C // 16, zrowd, 0)

    def zcp(k, _):
        pltpu.sync_copy(rows, acc_sh.at[pl.ds(s * RPT + k * C, C)])
        pltpu.sync_copy(exbuf, den_sh.at[pl.ds(s * RPT + k * C, C)])
        return 0
    lax.fori_loop(0, RPT // C, zcp, 0)
    rem = RPT % C
    pltpu.sync_copy(rows.at[pl.ds(0, rem)],
                    acc_sh.at[pl.ds(s * RPT + (RPT // C) * C, rem)])
    pltpu.sync_copy(exbuf.at[pl.ds(0, rem)],
                    den_sh.at[pl.ds(s * RPT + (RPT // C) * C, rem)])
    plsc.subcore_barrier()

    def chunk(i, _):
        base = w * EPW + i * C
        pltpu.sync_copy(src_hbm.at[pl.ds(base, C)], sidx)
        pltpu.sync_copy(dst_hbm.at[pl.ds(base, C)], didx)
        cp1 = pltpu.async_copy(hp_hbm.at[sidx], rows, sem)
        cp2 = pltpu.async_copy(ssrc_hbm.at[sidx], sv, sem2)
        cp3 = pltpu.async_copy(sdst_hbm.at[didx], dv, sem3)
        cp1.wait()
        cp2.wait()
        cp3.wait()

        def group(g, _):
            lo = sv[pl.ds(g * 16, 16)] + dv[pl.ds(g * 16, 16)]
            ex = jnp.exp(jnp.maximum(lo, lo * NEG))
            exbuf[pl.ds(g * 16, 16)] = ex
            for j in range(16):
                e = g * 16 + j
                rows[e, :] = rows[e, :] * ex[j]
            return 0
        lax.fori_loop(0, C // 16, group, 0)

        pltpu.sync_copy(rows, acc_sh.at[didx], add=True)
        pltpu.sync_copy(exbuf, den_sh.at[didx], add=True)
        return 0
    lax.fori_loop(0, NCHUNK, chunk, 0)

    plsc.subcore_barrier()
    pltpu.sync_copy(acc_sh.at[pl.ds(s * RPT, RPT)],
                    out_acc.at[c, pl.ds(s * RPT, RPT)])
    pltpu.sync_copy(den_sh.at[pl.ds(s * RPT, RPT)],
                    out_den.at[c, pl.ds(s * RPT, RPT)])


def _mlp_kernel(pooled_ref, m1_ref, b1_ref, m2_ref, b2_ref, m3_ref, b3_ref,
                out_ref):
    dot = functools.partial(jnp.dot, precision=lax.Precision.HIGHEST,
                            preferred_element_type=jnp.float32)
    z = jnp.maximum(dot(pooled_ref[...], m1_ref[...]) + b1_ref[...], 0.0)
    z = jnp.maximum(dot(z, m2_ref[...]) + b2_ref[...], 0.0)
    out_ref[...] = dot(z, m3_ref[...]) + b3_ref[...]


def kernel(x, edge_index, batch, W0, Wh, a_src, a_dst, bias, gamma, beta,
           M1, b1, M2, b2, M3, b3):
    dot = jnp.dot
    src = edge_index[0]
    dst = edge_index[1]
    h = x
    for l in range(L):
        W = W0 if l == 0 else Wh[l - 1]
        hp = dot(h, W)
        s_src = dot(hp, a_src[l])
        s_dst = dot(hp, a_dst[l])
        acc, den = _edge_pass(src, dst, hp, s_src, s_dst)
        lo = s_src + s_dst
        exs = jnp.exp(jnp.maximum(lo, lo * NEG))
        num = acc[0, :N] + acc[1, :N] + exs[:, None] * hp
        dtot = den[0, :N] + den[1, :N] + exs
        out = num / (dtot + 1e-16)[:, None] + bias[l]
        mu = jnp.mean(out, axis=0)
        var = jnp.var(out, axis=0)
        h = jax.nn.relu((out - mu) / jnp.sqrt(var + 1e-5) * gamma[l] + beta[l])
    onehot = (batch[:, None] == jnp.arange(G, dtype=batch.dtype)[None, :])
    onehot = onehot.astype(jnp.float32)
    sums = dot(onehot.T, h)
    cnt = jnp.sum(onehot, axis=0)
    pooled = sums / jnp.maximum(cnt, 1.0)[:, None]
    out = pl.pallas_call(
        _mlp_kernel,
        out_shape=jax.ShapeDtypeStruct((G, OUT), jnp.float32),
    )(pooled, M1, b1, M2, b2, M3, b3)
    return out
